# baseline (speedup 1.0000x reference)
import functools

import jax
import jax.numpy as jnp
from jax import lax
from jax.experimental import pallas as pl
from jax.experimental.pallas import tpu as pltpu
from jax.experimental.pallas import tpu_sc as plsc

_N = 10000
_D = 128
_DW = _D // 2
_E = 320000

_NC = 2
_NS = 16
_NW = _NC * _NS
_EC = 80
_CHUNKS = _E // _EC
_CPW = _CHUNKS // _NW
_NPAD = 10112
_ROWS_PER_TILE = _NPAD // _NS
_IRING = 8


def _mm_body(x_ref, w1_ref, w2_ref, b_ref, h1_ref, h2_ref):
    xb = x_ref[...]
    h1 = jnp.dot(xb, w1_ref[...], preferred_element_type=jnp.float32)
    h2 = (jnp.dot(xb, w2_ref[...], preferred_element_type=jnp.float32)
          + b_ref[...])
    h1_ref[...] = h1
    h2_ref[...] = h2


_MM_BLK = 1000


def _node_transform(x, w1, w2, b2d):
    grid = (_N // _MM_BLK,)
    return pl.pallas_call(
        _mm_body,
        grid=grid,
        in_specs=[
            pl.BlockSpec((_MM_BLK, _D), lambda i: (i, 0)),
            pl.BlockSpec((_D, _D), lambda i: (0, 0)),
            pl.BlockSpec((_D, _D), lambda i: (0, 0)),
            pl.BlockSpec((1, _D), lambda i: (0, 0)),
        ],
        out_specs=[
            pl.BlockSpec((_MM_BLK, _D), lambda i: (i, 0)),
            pl.BlockSpec((_MM_BLK, _D), lambda i: (i, 0)),
        ],
        out_shape=[
            jax.ShapeDtypeStruct((_N, _D), jnp.float32),
            jax.ShapeDtypeStruct((_N, _D), jnp.float32),
        ],
    )(x, w1, w2, b2d)


def _edge_body(h1_hbm, h2_hbm, src_hbm, dst_hbm, out0_hbm, out1_hbm,
               isb, idb, b1a, b2a, b1b, b2b, acc,
               sia, sib, sga1, sga2, sgb1, sgb2):
    cid = lax.axis_index("c")
    sid = lax.axis_index("s")
    wid = sid * _NC + cid
    k0 = wid * _CPW

    def _idx_issue(k, sem):
        sl = lax.rem(k - k0, _IRING)
        pltpu.async_copy(src_hbm.at[pl.ds(k * _EC, _EC)], isb.at[sl], sem)
        pltpu.async_copy(dst_hbm.at[pl.ds(k * _EC, _EC)], idb.at[sl], sem)

    def _idx_wait(k, sem):
        sl = lax.rem(k - k0, _IRING)
        pltpu.make_async_copy(src_hbm.at[pl.ds(k * _EC, _EC)], isb.at[sl], sem).wait()
        pltpu.make_async_copy(dst_hbm.at[pl.ds(k * _EC, _EC)], idb.at[sl], sem).wait()

    def _g_issue(k, b1, b2, s1, s2):
        sl = lax.rem(k - k0, _IRING)
        pltpu.async_copy(h1_hbm.at[isb.at[sl]], b1, s1)
        pltpu.async_copy(h2_hbm.at[idb.at[sl]], b2, s2)

    def _g_wait(k, b1, b2, s1, s2):
        sl = lax.rem(k - k0, _IRING)
        pltpu.make_async_copy(h1_hbm.at[isb.at[sl]], b1, s1).wait()
        pltpu.make_async_copy(h2_hbm.at[idb.at[sl]], b2, s2).wait()

    def _compute(b1, b2):
        def _row(r, _):
            for rr in range(5):
                ri = r * 5 + rr
                for j in range(_D // 16):
                    s = pl.ds(j * 16, 16)
                    b1[ri, s] = jnp.maximum(b1[ri, s] + b2[ri, s], 0.0)
            return 0

        lax.fori_loop(0, _EC // 5, _row, 0)

    def _scatter(k, b1):
        sl = lax.rem(k - k0, _IRING)
        pltpu.sync_copy(b1, acc.at[idb.at[sl]], add=True)

    _idx_issue(k0, sia)
    _idx_issue(k0 + 1, sib)

    zeros = jnp.zeros((16,), jnp.float32)

    def _zrow(r, _):
        for j in range(_D // 16):
            b1a[r, pl.ds(j * 16, 16)] = zeros
        return 0

    lax.fori_loop(0, _EC, _zrow, 0)

    arow = sid * _ROWS_PER_TILE
    for k in range(7):
        pltpu.sync_copy(b1a, acc.at[pl.ds(arow + k * _EC, _EC)])
    pltpu.sync_copy(b1a.at[pl.ds(0, 72)], acc.at[pl.ds(arow + 560, 72)])
    plsc.subcore_barrier()

    _idx_wait(k0, sia)
    _g_issue(k0, b1a, b2a, sga1, sga2)
    _idx_issue(k0 + 2, sia)
    _idx_wait(k0 + 1, sib)
    _g_issue(k0 + 1, b1b, b2b, sgb1, sgb2)
    _idx_issue(k0 + 3, sib)

    def _phase(k, b1, b2, s1, s2, si):
        _g_wait(k, b1, b2, s1, s2)
        _compute(b1, b2)
        _scatter(k, b1)

        @pl.when(k + 2 < k0 + _CPW)
        def _():
            _idx_wait(k + 2, si)
            _g_issue(k + 2, b1, b2, s1, s2)

        @pl.when(k + 4 < k0 + _CPW)
        def _():
            _idx_issue(k + 4, si)

    def _step(t, _):
        _phase(k0 + 2 * t, b1a, b2a, sga1, sga2, sia)
        _phase(k0 + 2 * t + 1, b1b, b2b, sgb1, sgb2, sib)
        return 0

    lax.fori_loop(0, _CPW // 2, _step, 0)
    _phase(k0 + _CPW - 1, b1a, b2a, sga1, sga2, sia)
    plsc.subcore_barrier()

    @pl.when(cid == 0)
    def _():
        pltpu.sync_copy(acc.at[pl.ds(arow, _ROWS_PER_TILE)],
                        out0_hbm.at[pl.ds(arow, _ROWS_PER_TILE)])

    @pl.when(cid == 1)
    def _():
        pltpu.sync_copy(acc.at[pl.ds(arow, _ROWS_PER_TILE)],
                        out1_hbm.at[pl.ds(arow, _ROWS_PER_TILE)])


def _edge_pass(h1, h2, src, dst):
    mesh = plsc.VectorSubcoreMesh(core_axis_name="c", subcore_axis_name="s")
    f = functools.partial(
        pl.kernel,
        mesh=mesh,
        compiler_params=pltpu.CompilerParams(needs_layout_passes=False),
        out_type=[
            jax.ShapeDtypeStruct((_NPAD, _D), jnp.float32),
            jax.ShapeDtypeStruct((_NPAD, _D), jnp.float32),
        ],
        scratch_types=[
            pltpu.VMEM((_IRING, _EC), jnp.int32),
            pltpu.VMEM((_IRING, _EC), jnp.int32),
            pltpu.VMEM((_EC, _D), jnp.float32),
            pltpu.VMEM((_EC, _D), jnp.float32),
            pltpu.VMEM((_EC, _D), jnp.float32),
            pltpu.VMEM((_EC, _D), jnp.float32),
            pltpu.VMEM_SHARED((_NPAD, _D), jnp.float32),
            pltpu.SemaphoreType.DMA,
            pltpu.SemaphoreType.DMA,
            pltpu.SemaphoreType.DMA,
            pltpu.SemaphoreType.DMA,
            pltpu.SemaphoreType.DMA,
            pltpu.SemaphoreType.DMA,
        ],
    )(_edge_body)
    return f(h1, h2, src, dst)


def _add_body(p_ref, q_ref, o_ref):
    o_ref[...] = p_ref[...] + q_ref[...]


_ADD_BLK = 80


def _final_add(p0, p1):
    grid = (_N // _ADD_BLK,)
    return pl.pallas_call(
        _add_body,
        grid=grid,
        in_specs=[
            pl.BlockSpec((_ADD_BLK, _D), lambda i: (i, 0)),
            pl.BlockSpec((_ADD_BLK, _D), lambda i: (i, 0)),
        ],
        out_specs=pl.BlockSpec((_ADD_BLK, _D), lambda i: (i, 0)),
        out_shape=jax.ShapeDtypeStruct((_N, _D), jnp.float32),
    )(p0, p1)


def kernel(x, edge_index, W, b):
    w1 = W[:_D]
    w2 = W[_D:]
    b2d = b.reshape(1, _D)
    h1, h2 = _node_transform(x, w1, w2, b2d)
    src = edge_index[0]
    dst = edge_index[1]
    p0, p1 = _edge_pass(h1, h2, src, dst)
    return _final_add(p0, p1)


# speedup vs baseline: 1.2250x; 1.2250x over previous
import functools

import jax
import jax.numpy as jnp
from jax import lax
from jax.experimental import pallas as pl
from jax.experimental.pallas import tpu as pltpu
from jax.experimental.pallas import tpu_sc as plsc

_N = 10000
_D = 128
_DW = _D // 2
_E = 320000

_NC = 2
_NS = 16
_NW = _NC * _NS
_EC = 80
_CHUNKS = _E // _EC
_CPW = _CHUNKS // _NW
_NPAD = 10112
_ROWS_PER_TILE = _NPAD // _NS
_IRING = 8


def _mm_body(x_ref, w1_ref, w2_ref, b_ref, h1_ref, h2_ref):
    xb = x_ref[...]
    h1 = jnp.dot(xb, w1_ref[...], preferred_element_type=jnp.float32)
    h2 = (jnp.dot(xb, w2_ref[...], preferred_element_type=jnp.float32)
          + b_ref[...])
    h1_ref[...] = h1
    h2_ref[...] = h2


_MM_BLK = 1000


def _node_transform(x, w1, w2, b2d):
    grid = (_N // _MM_BLK,)
    return pl.pallas_call(
        _mm_body,
        grid=grid,
        in_specs=[
            pl.BlockSpec((_MM_BLK, _D), lambda i: (i, 0)),
            pl.BlockSpec((_D, _D), lambda i: (0, 0)),
            pl.BlockSpec((_D, _D), lambda i: (0, 0)),
            pl.BlockSpec((1, _D), lambda i: (0, 0)),
        ],
        out_specs=[
            pl.BlockSpec((_MM_BLK, _D), lambda i: (i, 0)),
            pl.BlockSpec((_MM_BLK, _D), lambda i: (i, 0)),
        ],
        out_shape=[
            jax.ShapeDtypeStruct((_N, _D), jnp.float32),
            jax.ShapeDtypeStruct((_N, _D), jnp.float32),
        ],
    )(x, w1, w2, b2d)


def _edge_body(h1_hbm, h2_hbm, src_hbm, dst_hbm, out0_hbm, out1_hbm,
               isb, idb, b1a, b2a, b1b, b2b, acc,
               sia, sib, sga1, sga2, sgb1, sgb2):
    cid = lax.axis_index("c")
    sid = lax.axis_index("s")
    wid = sid * _NC + cid
    k0 = wid * _CPW

    def _idx_issue(k, sem):
        sl = lax.rem(k - k0, _IRING)
        pltpu.async_copy(src_hbm.at[pl.ds(k * _EC, _EC)], isb.at[sl], sem)
        pltpu.async_copy(dst_hbm.at[pl.ds(k * _EC, _EC)], idb.at[sl], sem)

    def _idx_wait(k, sem):
        sl = lax.rem(k - k0, _IRING)
        pltpu.make_async_copy(src_hbm.at[pl.ds(k * _EC, _EC)], isb.at[sl], sem).wait()
        pltpu.make_async_copy(dst_hbm.at[pl.ds(k * _EC, _EC)], idb.at[sl], sem).wait()

    def _g_issue(k, b1, b2, s1, s2):
        sl = lax.rem(k - k0, _IRING)
        pltpu.async_copy(h1_hbm.at[isb.at[sl]], b1, s1)
        pltpu.async_copy(h2_hbm.at[idb.at[sl]], b2, s2)

    def _g_wait(k, b1, b2, s1, s2):
        sl = lax.rem(k - k0, _IRING)
        pltpu.make_async_copy(h1_hbm.at[isb.at[sl]], b1, s1).wait()
        pltpu.make_async_copy(h2_hbm.at[idb.at[sl]], b2, s2).wait()

    def _compute(b1, b2):
        def _row(r, _):
            for rr in range(5):
                ri = r * 5 + rr
                for j in range(_D // 16):
                    s = pl.ds(j * 16, 16)
                    b1[ri, s] = jnp.maximum(b1[ri, s] + b2[ri, s], 0.0)
            return 0

        lax.fori_loop(0, _EC // 5, _row, 0)

    def _scatter(k, b1):
        sl = lax.rem(k - k0, _IRING)
        pltpu.sync_copy(b1, acc.at[idb.at[sl]], add=True)

    _idx_issue(k0, sia)
    _idx_issue(k0 + 1, sib)

    zeros = jnp.zeros((16,), jnp.float32)

    def _zrow(r, _):
        for j in range(_D // 16):
            b1a[r, pl.ds(j * 16, 16)] = zeros
        return 0

    lax.fori_loop(0, _EC, _zrow, 0)

    arow = sid * _ROWS_PER_TILE
    for k in range(7):
        pltpu.sync_copy(b1a, acc.at[pl.ds(arow + k * _EC, _EC)])
    pltpu.sync_copy(b1a.at[pl.ds(0, 72)], acc.at[pl.ds(arow + 560, 72)])
    plsc.subcore_barrier()

    _idx_wait(k0, sia)
    _g_issue(k0, b1a, b2a, sga1, sga2)
    _idx_issue(k0 + 2, sia)
    _idx_wait(k0 + 1, sib)
    _g_issue(k0 + 1, b1b, b2b, sgb1, sgb2)
    _idx_issue(k0 + 3, sib)

    def _phase(k, b1, b2, s1, s2, si):
        _g_wait(k, b1, b2, s1, s2)
        _compute(b1, b2)
        _scatter(k, b1)

        @pl.when(k + 2 < k0 + _CPW)
        def _():
            _idx_wait(k + 2, si)
            _g_issue(k + 2, b1, b2, s1, s2)

        @pl.when(k + 4 < k0 + _CPW)
        def _():
            _idx_issue(k + 4, si)

    def _step(t, _):
        _phase(k0 + 2 * t, b1a, b2a, sga1, sga2, sia)
        _phase(k0 + 2 * t + 1, b1b, b2b, sgb1, sgb2, sib)
        return 0

    lax.fori_loop(0, _CPW // 2, _step, 0)
    _phase(k0 + _CPW - 1, b1a, b2a, sga1, sga2, sia)
    plsc.subcore_barrier()

    @pl.when(cid == 0)
    def _():
        pltpu.sync_copy(acc.at[pl.ds(arow, _ROWS_PER_TILE)],
                        out0_hbm.at[pl.ds(arow, _ROWS_PER_TILE)])

    @pl.when(cid == 1)
    def _():
        pltpu.sync_copy(acc.at[pl.ds(arow, _ROWS_PER_TILE)],
                        out1_hbm.at[pl.ds(arow, _ROWS_PER_TILE)])


def _edge_pass(h1, h2, src, dst):
    mesh = plsc.VectorSubcoreMesh(core_axis_name="c", subcore_axis_name="s")
    f = functools.partial(
        pl.kernel,
        mesh=mesh,
        out_type=[
            jax.ShapeDtypeStruct((_NPAD, _D), jnp.float32),
            jax.ShapeDtypeStruct((_NPAD, _D), jnp.float32),
        ],
        scratch_types=[
            pltpu.VMEM((_IRING, _EC), jnp.int32),
            pltpu.VMEM((_IRING, _EC), jnp.int32),
            pltpu.VMEM((_EC, _D), jnp.float32),
            pltpu.VMEM((_EC, _D), jnp.float32),
            pltpu.VMEM((_EC, _D), jnp.float32),
            pltpu.VMEM((_EC, _D), jnp.float32),
            pltpu.VMEM_SHARED((_NPAD, _D), jnp.float32),
            pltpu.SemaphoreType.DMA,
            pltpu.SemaphoreType.DMA,
            pltpu.SemaphoreType.DMA,
            pltpu.SemaphoreType.DMA,
            pltpu.SemaphoreType.DMA,
            pltpu.SemaphoreType.DMA,
        ],
    )(_edge_body)
    return f(h1, h2, src, dst)


def _add_body(p_ref, q_ref, o_ref):
    o_ref[...] = p_ref[...] + q_ref[...]


_ADD_BLK = 1000


def _final_add(p0, p1):
    grid = (_N // _ADD_BLK,)
    return pl.pallas_call(
        _add_body,
        grid=grid,
        in_specs=[
            pl.BlockSpec((_ADD_BLK, _D), lambda i: (i, 0)),
            pl.BlockSpec((_ADD_BLK, _D), lambda i: (i, 0)),
        ],
        out_specs=pl.BlockSpec((_ADD_BLK, _D), lambda i: (i, 0)),
        out_shape=jax.ShapeDtypeStruct((_N, _D), jnp.float32),
    )(p0, p1)


def kernel(x, edge_index, W, b):
    w1 = W[:_D]
    w2 = W[_D:]
    b2d = b.reshape(1, _D)
    h1, h2 = _node_transform(x, w1, w2, b2d)
    src = edge_index[0]
    dst = edge_index[1]
    p0, p1 = _edge_pass(h1, h2, src, dst)
    return _final_add(p0, p1)
